# Initial kernel scaffold; baseline (speedup 1.0000x reference)
import jax
import jax.numpy as jnp
from jax.experimental import pallas as pl


def kernel(user, item, user_table, item_table, W1, b1, W2, b2):
    raise NotImplementedError("write your pallas kernel here")


# speedup vs baseline: 3.3653x; 3.3653x over previous
import functools

import jax
import jax.numpy as jnp
from jax import lax
from jax.experimental import pallas as pl
from jax.experimental.pallas import tpu as pltpu
from jax.experimental.pallas import tpu_sc as plsc


def _make_sc_gather(V_u, V_i, B, D):
    info = plsc.get_sparse_core_info()
    nw = info.num_cores * info.num_subcores
    b_per_w = B // nw
    assert B % (8 * nw) == 0
    mesh = plsc.VectorSubcoreMesh(core_axis_name="c", subcore_axis_name="s")

    @functools.partial(
        pl.kernel,
        mesh=mesh,
        out_type=[
            jax.ShapeDtypeStruct((B, D), jnp.float32),
            jax.ShapeDtypeStruct((B, D), jnp.float32),
        ],
        scratch_types=[
            pltpu.VMEM((b_per_w,), jnp.int32),
            pltpu.VMEM((b_per_w, D), jnp.float32),
            pltpu.SemaphoreType.DMA,
        ],
    )
    def gather_k(u_table, i_table, u_idx, i_idx, u_out, i_out, idx_v, rows_v, sem):
        wid = lax.axis_index("s") * info.num_cores + lax.axis_index("c")
        base = wid * b_per_w
        pltpu.sync_copy(u_idx.at[pl.ds(base, b_per_w)], idx_v)
        pltpu.async_copy(u_table.at[idx_v], rows_v, sem).wait()
        pltpu.sync_copy(rows_v, u_out.at[pl.ds(base, b_per_w)])
        pltpu.sync_copy(i_idx.at[pl.ds(base, b_per_w)], idx_v)
        pltpu.async_copy(i_table.at[idx_v], rows_v, sem).wait()
        pltpu.sync_copy(rows_v, i_out.at[pl.ds(base, b_per_w)])

    return gather_k


def _mlp_body(u_ref, i_ref, w1a_ref, w1b_ref, b1_ref, w2_ref, b2_ref, o_ref):
    h = jnp.dot(u_ref[...], w1a_ref[...], preferred_element_type=jnp.float32)
    h += jnp.dot(i_ref[...], w1b_ref[...], preferred_element_type=jnp.float32)
    h = jnp.maximum(h + b1_ref[...], 0.0)
    o_ref[...] = (
        jnp.dot(h, w2_ref[...], preferred_element_type=jnp.float32) + b2_ref[...]
    )


def _mlp(u_emb, i_emb, W1a, W1b, b1, W2, b2, block_b=2048):
    B, D = u_emb.shape
    H = W1a.shape[1]
    return pl.pallas_call(
        _mlp_body,
        grid=(B // block_b,),
        in_specs=[
            pl.BlockSpec((block_b, D), lambda i: (i, 0)),
            pl.BlockSpec((block_b, D), lambda i: (i, 0)),
            pl.BlockSpec((D, H), lambda i: (0, 0)),
            pl.BlockSpec((D, H), lambda i: (0, 0)),
            pl.BlockSpec((1, H), lambda i: (0, 0)),
            pl.BlockSpec((H, D), lambda i: (0, 0)),
            pl.BlockSpec((1, D), lambda i: (0, 0)),
        ],
        out_specs=pl.BlockSpec((block_b, D), lambda i: (i, 0)),
        out_shape=jax.ShapeDtypeStruct((B, D), jnp.float32),
    )(u_emb, i_emb, W1a, W1b, b1.reshape(1, H), W2, b2.reshape(1, D))


def kernel(user, item, user_table, item_table, W1, b1, W2, b2):
    B = user.shape[0]
    V_u, D = user_table.shape
    V_i = item_table.shape[0]
    gather = _make_sc_gather(V_u, V_i, B, D)
    u_emb, i_emb = gather(
        user_table, item_table, user.astype(jnp.int32), item.astype(jnp.int32)
    )
    W1a = W1[:D]
    W1b = W1[D:]
    return _mlp(u_emb, i_emb, W1a, W1b, b1, W2, b2)
